# baseline (speedup 1.0000x reference)
import functools

import jax
import jax.numpy as jnp
from jax import lax
from jax.experimental import pallas as pl
from jax.experimental.pallas import tpu as pltpu
from jax.experimental.pallas import tpu_sc as plsc

B, T = 16384, 100
D = 128
N = B * T
NUM_ROWS = 5
NC, NS = 2, 16
NW = NC * NS
PER_W = N // NW
CHUNK = 128
NCHUNK = PER_W // CHUNK
NBUF = 4

BLK = 8192
W = BLK // 8
NBLK = N // BLK


@functools.partial(
    pl.kernel,
    mesh=plsc.VectorSubcoreMesh(core_axis_name="c", subcore_axis_name="s"),
    out_type=jax.ShapeDtypeStruct((N, D), jnp.float32),
    scratch_types=[
        pltpu.VMEM_SHARED((NUM_ROWS, D), jnp.float32),
        pltpu.VMEM((NCHUNK, CHUNK), jnp.int32),
        pltpu.VMEM((NBUF, CHUNK, D), jnp.float32),
        pltpu.SemaphoreType.DMA,
        pltpu.SemaphoreType.DMA,
    ],
)
def _sc_gather(idx_hbm, table_hbm, out_hbm, tab_s, idx_v, rows_v, gsem, ssem):
    cid = lax.axis_index("c")
    sid = lax.axis_index("s")
    wid = sid * NC + cid
    base = wid * PER_W

    @pl.when(sid == 0)
    def _():
        pltpu.sync_copy(table_hbm, tab_s)

    plsc.subcore_barrier()
    pltpu.sync_copy(idx_hbm.at[wid], idx_v)
    pltpu.async_copy(tab_s.at[idx_v.at[0]], rows_v.at[0], gsem)

    def body(p, carry):
        for b in range(NBUF):
            g = p * NBUF + b
            nb = (b + 1) % NBUF

            @pl.when(g + 1 < NCHUNK)
            def _():
                @pl.when(g + 1 >= NBUF)
                def _():
                    off_r = base + (g + 1 - NBUF) * CHUNK
                    pltpu.make_async_copy(
                        rows_v.at[nb], out_hbm.at[pl.ds(off_r, CHUNK)], ssem
                    ).wait()

                pltpu.async_copy(tab_s.at[idx_v.at[g + 1]], rows_v.at[nb], gsem)

            pltpu.make_async_copy(tab_s.at[idx_v.at[g]], rows_v.at[b], gsem).wait()
            pltpu.async_copy(
                rows_v.at[b], out_hbm.at[pl.ds(base + g * CHUNK, CHUNK)], ssem
            )
        return carry

    lax.fori_loop(0, NCHUNK // NBUF, body, 0)

    for b in range(NBUF):
        pltpu.make_async_copy(
            rows_v.at[b], out_hbm.at[pl.ds(base, CHUNK)], ssem
        ).wait()


def _tc_body(tok_ref, table_ref, out_ref):
    for s in range(8):
        tok = jnp.reshape(tok_ref[0, s, :], (W, 1))
        acc = jnp.broadcast_to(table_ref[0][None, :], (W, D))
        for k in range(1, NUM_ROWS):
            acc = jnp.where(tok == k, table_ref[k][None, :], acc)
        out_ref[pl.ds(s * W, W), :] = acc


def _tc_gather(idx3, table):
    return pl.pallas_call(
        _tc_body,
        grid=(NBLK,),
        in_specs=[
            pl.BlockSpec((1, 8, W), lambda i: (i, 0, 0)),
            pl.BlockSpec((NUM_ROWS, D), lambda i: (0, 0)),
        ],
        out_specs=pl.BlockSpec((BLK, D), lambda i: (i, 0)),
        out_shape=jax.ShapeDtypeStruct((N, D), jnp.float32),
    )(idx3, table)


def kernel(token_types, table):
    idx_sc = jnp.reshape(token_types, (NW, NCHUNK, CHUNK)).astype(jnp.int32)
    idx_tc = jnp.reshape(token_types, (NBLK, 8, W)).astype(jnp.int32)
    sc_out = _sc_gather(idx_sc, table)
    tc_out = _tc_gather(idx_tc, table)
    out = lax.dynamic_update_slice(
        tc_out, lax.slice(sc_out, (0, 0), (1, 1)), (0, 0)
    )
    return jnp.reshape(out, (B, T, D))
